# Initial kernel scaffold; baseline (speedup 1.0000x reference)
import jax
import jax.numpy as jnp
from jax.experimental import pallas as pl


def kernel(inputs, params):
    raise NotImplementedError("write your pallas kernel here")


# speedup vs baseline: 1.9910x; 1.9910x over previous
import functools

import jax
import jax.numpy as jnp
from jax import lax
from jax.experimental import pallas as pl
from jax.experimental.pallas import tpu as pltpu

SEQ = 2048
EMB = 768
HEADS = 12
DIMH = 64
NHASH = 4
NBUCKETS = 32
CHUNK = 64
NCHUNK = NHASH * NBUCKETS
TOTAL = NHASH * SEQ
NITEM = HEADS * TOTAL
NTOK = 32000

_INTERPRET = False


def _pcall(body, **kw):
    return pl.pallas_call(body, interpret=_INTERPRET, **kw)


def _layer_norm(x, g, b):
    mu = jnp.mean(x, axis=-1, keepdims=True)
    var = jnp.mean(jnp.square(x - mu), axis=-1, keepdims=True)
    return (x - mu) / jnp.sqrt(var + 1e-3) * g + b


def _qkv_body(x_ref, g_ref, b_ref, wqk_ref, wv_ref, qk_ref, v_ref):
    xn = _layer_norm(x_ref[...], g_ref[...], b_ref[...])
    qk_ref[...] = jnp.dot(xn, wqk_ref[...], preferred_element_type=jnp.float32)
    v_ref[...] = jnp.dot(xn, wv_ref[...], preferred_element_type=jnp.float32)


def _qkv(x, g, b, wqk, wv):
    blk = 256
    grid = (SEQ // blk,)
    return _pcall(
        _qkv_body,
        grid=grid,
        in_specs=[
            pl.BlockSpec((blk, EMB), lambda i: (i, 0)),
            pl.BlockSpec((EMB,), lambda i: (0,)),
            pl.BlockSpec((EMB,), lambda i: (0,)),
            pl.BlockSpec((EMB, EMB), lambda i: (0, 0)),
            pl.BlockSpec((EMB, EMB), lambda i: (0, 0)),
        ],
        out_specs=[
            pl.BlockSpec((blk, EMB), lambda i: (i, 0)),
            pl.BlockSpec((blk, EMB), lambda i: (i, 0)),
        ],
        out_shape=[
            jax.ShapeDtypeStruct((SEQ, EMB), jnp.float32),
            jax.ShapeDtypeStruct((SEQ, EMB), jnp.float32),
        ],
    )(x, g, b, wqk, wv)


def _dest_body(qk_ref, rot_ref, dest_ref):
    hh = pl.program_id(0)
    nb2 = NBUCKETS
    blk = 256
    nblk = SEQ // blk
    ii = lax.broadcasted_iota(jnp.int32, (blk, blk), 0)
    jj = lax.broadcasted_iota(jnp.int32, (blk, blk), 1)
    tri = (ii >= jj).astype(jnp.float32)
    bi = lax.broadcasted_iota(jnp.int32, (nb2, nb2), 0)
    bj = lax.broadcasted_iota(jnp.int32, (nb2, nb2), 1)
    up = (bi < bj).astype(jnp.float32)
    for hd in range(HEADS):
        qh = qk_ref[:, hd * DIMH:(hd + 1) * DIMH]
        r = jnp.dot(qh, rot_ref[0], preferred_element_type=jnp.float32)
        vals = jnp.concatenate([r, -r], axis=1)
        mx = jnp.max(vals, axis=1, keepdims=True)
        iota = lax.broadcasted_iota(jnp.int32, vals.shape, 1)
        b = jnp.min(jnp.where(vals >= mx, iota, nb2), axis=1)
        onehot = (b[:, None] == lax.broadcasted_iota(jnp.int32, (SEQ, nb2), 1))
        onehot = onehot.astype(jnp.float32)
        carry = jnp.zeros((1, nb2), jnp.float32)
        ranks = []
        for k in range(nblk):
            ob = onehot[k * blk:(k + 1) * blk]
            pc = jnp.dot(tri, ob, preferred_element_type=jnp.float32) + carry
            ranks.append(jnp.sum(pc * ob, axis=1) - 1.0)
            carry = carry + jnp.sum(ob, axis=0, keepdims=True)
        rank = jnp.concatenate(ranks, axis=0)
        offs = jnp.dot(carry, up, preferred_element_type=jnp.float32)
        offs_b = jnp.sum(onehot * offs, axis=1)
        base = hd * TOTAL + hh * SEQ
        dest_ref[0, 0, pl.ds(hd * SEQ, SEQ)] = (offs_b + rank).astype(jnp.int32) + base


def _dest(qk, rot2):
    return _pcall(
        _dest_body,
        grid=(NHASH,),
        in_specs=[
            pl.BlockSpec((SEQ, EMB), lambda hh: (0, 0)),
            pl.BlockSpec((1, DIMH, NBUCKETS // 2), lambda hh: (hh, 0, 0)),
        ],
        out_specs=pl.BlockSpec((1, 1, HEADS * SEQ), lambda hh: (hh, 0, 0)),
        out_shape=jax.ShapeDtypeStruct((NHASH, 1, HEADS * SEQ), jnp.int32),
    )(qk, rot2)


_CB = 16


def _attn_body(qc_ref, qp_ref, vc_ref, vp_ref, pc_ref, pp_ref, so_ref, lg_ref):
    scale = float(DIMH) ** -0.5
    for i in range(_CB):
        q = qc_ref[0, i]
        qprev = qc_ref[0, i - 1] if i > 0 else qp_ref[0, _CB - 1]
        vcur = vc_ref[0, i]
        vprev = vc_ref[0, i - 1] if i > 0 else vp_ref[0, _CB - 1]
        posq = pc_ref[0, i][:, 0:1]
        posp = (pc_ref[0, i - 1] if i > 0 else pp_ref[0, _CB - 1])[:, 0:1]
        kc = q / (jnp.sqrt(jnp.sum(q * q, axis=1, keepdims=True)) + 1e-8)
        kp = qprev / (jnp.sqrt(jnp.sum(qprev * qprev, axis=1, keepdims=True)) + 1e-8)
        dn = (((1,), (1,)), ((), ()))
        dc = lax.dot_general(q, kc, dn, preferred_element_type=jnp.float32)
        dp = lax.dot_general(q, kp, dn, preferred_element_type=jnp.float32)
        dots = jnp.concatenate([dc, dp], axis=1) * scale
        keypos = jnp.concatenate([posq, posp], axis=0)
        mask = posq == keypos.T
        dots = jnp.where(mask, dots - 1e5, dots)
        m = jnp.max(dots, axis=1, keepdims=True)
        e = jnp.exp(dots - m)
        s = jnp.sum(e, axis=1, keepdims=True)
        lse = m + jnp.log(s)
        bv = jnp.concatenate([vcur, vprev], axis=0)
        bo = jnp.dot(e / s, bv, preferred_element_type=jnp.float32)
        so_ref[0, i] = bo
        lg_ref[0, i] = jnp.broadcast_to(lse, (CHUNK, 16))


def _attention(sqk, sv, spos):
    nb = NCHUNK // _CB
    grid = (HEADS, nb)

    def cur(hd, cb):
        return (hd, cb, 0, 0)

    def prev(hd, cb):
        return (hd, (cb + nb - 1) % nb, 0, 0)

    return _pcall(
        _attn_body,
        grid=grid,
        in_specs=[
            pl.BlockSpec((1, _CB, CHUNK, DIMH), cur),
            pl.BlockSpec((1, _CB, CHUNK, DIMH), prev),
            pl.BlockSpec((1, _CB, CHUNK, DIMH), cur),
            pl.BlockSpec((1, _CB, CHUNK, DIMH), prev),
            pl.BlockSpec((1, _CB, CHUNK, 16), cur),
            pl.BlockSpec((1, _CB, CHUNK, 16), prev),
        ],
        out_specs=[
            pl.BlockSpec((1, _CB, CHUNK, DIMH), cur),
            pl.BlockSpec((1, _CB, CHUNK, 16), cur),
        ],
        out_shape=[
            jax.ShapeDtypeStruct((HEADS, NCHUNK, CHUNK, DIMH), jnp.float32),
            jax.ShapeDtypeStruct((HEADS, NCHUNK, CHUNK, 16), jnp.float32),
        ],
    )(sqk, sqk, sv, sv, spos, spos)


def _combine_body(og_ref, lg_ref, x1_ref, x2_ref, wo_ref, bo_ref,
                  gg_ref, gb_ref, w1_ref, b1_ref, w2_ref, b2_ref,
                  y1_ref, y2_ref):
    l16 = lg_ref[...]
    m = jnp.max(l16, axis=0, keepdims=True)
    e = jnp.exp(l16 - m)
    probs16 = e / jnp.sum(e, axis=0, keepdims=True)
    ir = lax.broadcasted_iota(jnp.int32, (HEADS * 16, EMB), 0)
    ic = lax.broadcasted_iota(jnp.int32, (HEADS * 16, EMB), 1)
    ex = jnp.logical_and(ic // DIMH == ir // 16, ir % 16 == 0).astype(jnp.float32)
    acc = jnp.zeros_like(x1_ref[...])
    for h in range(NHASH):
        ph = jnp.dot(probs16[h], ex, preferred_element_type=jnp.float32)
        acc = acc + og_ref[h] * ph
    attn = jnp.dot(acc, wo_ref[...], preferred_element_type=jnp.float32) + bo_ref[...]
    y1 = x1_ref[...] + attn
    y1_ref[...] = y1
    xg = _layer_norm(y1, gg_ref[...], gb_ref[...])
    hmid = jax.nn.gelu(jnp.dot(xg, w1_ref[...], preferred_element_type=jnp.float32)
                       + b1_ref[...])
    y2_ref[...] = x2_ref[...] + jnp.dot(hmid, w2_ref[...],
                                        preferred_element_type=jnp.float32) + b2_ref[...]


def _combine_ffn(og, lg, x1, x2, lp):
    blk = 256
    grid = (SEQ // blk,)
    full = lambda shape: pl.BlockSpec(shape, lambda i: tuple(0 for _ in shape))
    return _pcall(
        _combine_body,
        grid=grid,
        in_specs=[
            pl.BlockSpec((NHASH, blk, EMB), lambda i: (0, i, 0)),
            pl.BlockSpec((NHASH, blk, HEADS * 16), lambda i: (0, i, 0)),
            pl.BlockSpec((blk, EMB), lambda i: (i, 0)),
            pl.BlockSpec((blk, EMB), lambda i: (i, 0)),
            full((EMB, EMB)),
            full((EMB,)),
            full((EMB,)),
            full((EMB,)),
            full((EMB, 4 * EMB)),
            full((4 * EMB,)),
            full((4 * EMB, EMB)),
            full((EMB,)),
        ],
        out_specs=[
            pl.BlockSpec((blk, EMB), lambda i: (i, 0)),
            pl.BlockSpec((blk, EMB), lambda i: (i, 0)),
        ],
        out_shape=[
            jax.ShapeDtypeStruct((SEQ, EMB), jnp.float32),
            jax.ShapeDtypeStruct((SEQ, EMB), jnp.float32),
        ],
    )(og, lg, x1, x2, lp['wo'], lp['bo'], lp['g_g'], lp['g_b'],
      lp['w1'], lp['b1'], lp['w2'], lp['b2'])


def _logits_body(x1_ref, x2_ref, w_ref, b_ref, out_ref):
    x = x1_ref[...] + x2_ref[...]
    out_ref[...] = (jnp.dot(x, w_ref[...], preferred_element_type=jnp.float32)
                    + b_ref[...])


def _logits(x1, x2, w, b):
    nt = 1280
    grid = (NTOK // nt,)
    return _pcall(
        _logits_body,
        grid=grid,
        in_specs=[
            pl.BlockSpec((SEQ, EMB), lambda i: (0, 0)),
            pl.BlockSpec((SEQ, EMB), lambda i: (0, 0)),
            pl.BlockSpec((EMB, nt), lambda i: (0, i)),
            pl.BlockSpec((1, nt), lambda i: (0, i)),
        ],
        out_specs=pl.BlockSpec((SEQ, nt), lambda i: (0, i)),
        out_shape=jax.ShapeDtypeStruct((SEQ, NTOK), jnp.float32),
    )(x1, x2, w, b.reshape(1, NTOK))


def _embed_add_body(e_ref, p_ref, o_ref):
    o_ref[...] = e_ref[...] + p_ref[...]


def _embed_add(e, p):
    return _pcall(
        _embed_add_body,
        grid=(SEQ // 256,),
        in_specs=[pl.BlockSpec((256, EMB), lambda i: (i, 0)),
                  pl.BlockSpec((256, EMB), lambda i: (i, 0))],
        out_specs=pl.BlockSpec((256, EMB), lambda i: (i, 0)),
        out_shape=jax.ShapeDtypeStruct((SEQ, EMB), jnp.float32),
    )(e, p)


def _scatter_sorted(qk, v, gdest):
    qk_h = qk.reshape(SEQ, HEADS, DIMH).transpose(1, 0, 2)
    v_h = v.reshape(SEQ, HEADS, DIMH).transpose(1, 0, 2)
    slot = gdest.reshape(HEADS, TOTAL) - (jnp.arange(HEADS) * TOTAL)[:, None]
    qk4 = jnp.broadcast_to(qk_h[:, None], (HEADS, NHASH, SEQ, DIMH)).reshape(HEADS, TOTAL, DIMH)
    v4 = jnp.broadcast_to(v_h[:, None], (HEADS, NHASH, SEQ, DIMH)).reshape(HEADS, TOTAL, DIMH)
    pos4 = jnp.broadcast_to(jnp.arange(SEQ, dtype=jnp.float32)[None, None, :],
                            (HEADS, NHASH, SEQ)).reshape(HEADS, TOTAL)
    hidx = jnp.arange(HEADS)[:, None]
    sqk = jnp.zeros((HEADS, TOTAL, DIMH), jnp.float32).at[hidx, slot].set(qk4)
    sv = jnp.zeros((HEADS, TOTAL, DIMH), jnp.float32).at[hidx, slot].set(v4)
    sp = jnp.zeros((HEADS, TOTAL), jnp.float32).at[hidx, slot].set(pos4)
    spos = jnp.broadcast_to(sp[..., None], (HEADS, TOTAL, 16))
    return sqk, sv, spos, slot


def _gather_unsorted(so, lg, slot):
    o4 = jnp.take_along_axis(so, slot[..., None], axis=1)
    l4 = jnp.take_along_axis(lg, slot[..., None], axis=1)
    og = o4.reshape(HEADS, NHASH, SEQ, DIMH).transpose(1, 2, 0, 3).reshape(NHASH, SEQ, EMB)
    lg16 = l4.reshape(HEADS, NHASH, SEQ, 16).transpose(1, 2, 0, 3).reshape(NHASH, SEQ, HEADS * 16)
    return og, lg16


def kernel(inputs, params):
    tokens = inputs.reshape(SEQ)
    emb = jnp.take(params['tok_emb'], tokens, axis=0)
    h = _embed_add(emb, params['pos_emb'])
    x1 = h
    x2 = h
    for lp in params['layers']:
        qk, v = _qkv(x2, lp['f_g'], lp['f_b'], lp['wqk'], lp['wv'])
        rot2 = lp['rot'].transpose(1, 0, 2)
        gdest = _dest(qk, rot2).reshape(NHASH, HEADS, SEQ).transpose(1, 0, 2)
        sqk, sv, spos, slot = _scatter_sorted(qk, v, gdest)
        so, lg = _attention(sqk.reshape(HEADS, NCHUNK, CHUNK, DIMH),
                            sv.reshape(HEADS, NCHUNK, CHUNK, DIMH),
                            spos.reshape(HEADS, NCHUNK, CHUNK, 16))
        og, lg16 = _gather_unsorted(so.reshape(HEADS, TOTAL, DIMH),
                                    lg.reshape(HEADS, TOTAL, 16), slot)
        x1, x2 = _combine_ffn(og, lg16, x1, x2, lp)
    out = _logits(x1, x2, params['w_logits'], params['b_logits'])
    return out.reshape(1, SEQ, NTOK)


# speedup vs baseline: 2.2509x; 1.1306x over previous
import functools

import jax
import jax.numpy as jnp
from jax import lax
from jax.experimental import pallas as pl
from jax.experimental.pallas import tpu as pltpu
from jax.experimental.pallas import tpu_sc as plsc

SEQ = 2048
EMB = 768
HEADS = 12
DIMH = 64
NHASH = 4
NBUCKETS = 32
CHUNK = 64
NCHUNK = NHASH * NBUCKETS
TOTAL = NHASH * SEQ
NITEM = HEADS * TOTAL
NTOK = 32000

_INTERPRET = False


def _pcall(body, **kw):
    return pl.pallas_call(body, interpret=_INTERPRET, **kw)


def _layer_norm(x, g, b):
    mu = jnp.mean(x, axis=-1, keepdims=True)
    var = jnp.mean(jnp.square(x - mu), axis=-1, keepdims=True)
    return (x - mu) / jnp.sqrt(var + 1e-3) * g + b


def _qkv_body(x_ref, wqk_ref, wv_ref, qk_ref, v_ref):
    xn = x_ref[...]
    qk_ref[...] = jnp.dot(xn, wqk_ref[...], preferred_element_type=jnp.float32)
    v_ref[...] = jnp.dot(xn, wv_ref[...], preferred_element_type=jnp.float32)


def _qkv(xn, wqk, wv):
    blk = 256
    grid = (SEQ // blk,)
    return _pcall(
        _qkv_body,
        grid=grid,
        in_specs=[
            pl.BlockSpec((blk, EMB), lambda i: (i, 0)),
            pl.BlockSpec((EMB, EMB), lambda i: (0, 0)),
            pl.BlockSpec((EMB, EMB), lambda i: (0, 0)),
        ],
        out_specs=[
            pl.BlockSpec((blk, EMB), lambda i: (i, 0)),
            pl.BlockSpec((blk, EMB), lambda i: (i, 0)),
        ],
        out_shape=[
            jax.ShapeDtypeStruct((SEQ, EMB), jnp.float32),
            jax.ShapeDtypeStruct((SEQ, EMB), jnp.float32),
        ],
    )(xn, wqk, wv)


def _dest_body(qk_ref, rot_ref, dest_ref):
    nb2 = NBUCKETS
    blk = 256
    nblk = SEQ // blk
    ii = lax.broadcasted_iota(jnp.int32, (blk, blk), 0)
    jj = lax.broadcasted_iota(jnp.int32, (blk, blk), 1)
    tri = (ii >= jj).astype(jnp.float32)
    bi = lax.broadcasted_iota(jnp.int32, (nb2, nb2), 0)
    bj = lax.broadcasted_iota(jnp.int32, (nb2, nb2), 1)
    up = (bi < bj).astype(jnp.float32)
    hh = pl.program_id(0)
    for hd in range(HEADS):
        qh = qk_ref[:, hd * DIMH:(hd + 1) * DIMH]
        r = jnp.dot(qh, rot_ref[0], preferred_element_type=jnp.float32)
        vals = jnp.concatenate([r, -r], axis=1)
        mx = jnp.max(vals, axis=1, keepdims=True)
        iota = lax.broadcasted_iota(jnp.int32, vals.shape, 1)
        b = jnp.min(jnp.where(vals >= mx, iota, nb2), axis=1)
        onehot = (b[:, None] == lax.broadcasted_iota(jnp.int32, (SEQ, nb2), 1))
        onehot = onehot.astype(jnp.float32)
        carry = jnp.zeros((1, nb2), jnp.float32)
        ranks = []
        for k in range(nblk):
            ob = onehot[k * blk:(k + 1) * blk]
            pc = jnp.dot(tri, ob, preferred_element_type=jnp.float32) + carry
            ranks.append(jnp.sum(pc * ob, axis=1) - 1.0)
            carry = carry + jnp.sum(ob, axis=0, keepdims=True)
        rank = jnp.concatenate(ranks, axis=0)
        offs = jnp.dot(carry, up, preferred_element_type=jnp.float32)
        offs_b = jnp.sum(onehot * offs, axis=1)
        base = hd * TOTAL + hh * SEQ
        dest_ref[0, 0, pl.ds(hd * SEQ, SEQ)] = (offs_b + rank).astype(jnp.int32) + base


def _dest(qk, rot2):
    return _pcall(
        _dest_body,
        grid=(NHASH,),
        in_specs=[
            pl.BlockSpec((SEQ, EMB), lambda hh: (0, 0)),
            pl.BlockSpec((1, DIMH, NBUCKETS // 2), lambda hh: (hh, 0, 0)),
        ],
        out_specs=pl.BlockSpec((1, 1, HEADS * SEQ), lambda hh: (hh, 0, 0)),
        out_shape=jax.ShapeDtypeStruct((NHASH, 1, HEADS * SEQ), jnp.int32),
    )(qk, rot2)


_CB = 16


def _attn_dots_body(qc_ref, kc_ref, kp_ref, pc_ref, pp_ref, dots_ref):
    scale = float(DIMH) ** -0.5
    dn = (((1,), (1,)), ((), ()))
    for i in range(_CB):
        q = qc_ref[0, i]
        kcur = kc_ref[0, i]
        kprev = kc_ref[0, i - 1] if i > 0 else kp_ref[0, _CB - 1]
        posq = pc_ref[0, i][:, 0:1]
        posp = (pc_ref[0, i - 1] if i > 0 else pp_ref[0, _CB - 1])[:, 0:1]
        dc = lax.dot_general(q, kcur, dn, preferred_element_type=jnp.float32)
        dp = lax.dot_general(q, kprev, dn, preferred_element_type=jnp.float32)
        dots = jnp.concatenate([dc, dp], axis=1) * scale
        keypos = jnp.concatenate([posq, posp], axis=0)
        mask = posq == keypos.T
        dots_ref[0, i] = jnp.where(mask, dots - 1e5, dots)


def _attn_dots(sqk, sk, spos):
    nb = NCHUNK // _CB
    grid = (HEADS, nb)

    def cur(hd, cb):
        return (hd, cb, 0, 0)

    def prev(hd, cb):
        return (hd, (cb + nb - 1) % nb, 0, 0)

    return _pcall(
        _attn_dots_body,
        grid=grid,
        in_specs=[
            pl.BlockSpec((1, _CB, CHUNK, DIMH), cur),
            pl.BlockSpec((1, _CB, CHUNK, DIMH), cur),
            pl.BlockSpec((1, _CB, CHUNK, DIMH), prev),
            pl.BlockSpec((1, _CB, CHUNK, 16), cur),
            pl.BlockSpec((1, _CB, CHUNK, 16), prev),
        ],
        out_specs=pl.BlockSpec((1, _CB, CHUNK, 2 * CHUNK), cur),
        out_shape=jax.ShapeDtypeStruct((HEADS, NCHUNK, CHUNK, 2 * CHUNK), jnp.float32),
    )(sqk, sk, sk, spos, spos)


def _attn_av_body(pr_ref, vc_ref, vp_ref, so_ref):
    for i in range(_CB):
        vcur = vc_ref[0, i]
        vprev = vc_ref[0, i - 1] if i > 0 else vp_ref[0, _CB - 1]
        bv = jnp.concatenate([vcur, vprev], axis=0)
        so_ref[0, i] = jnp.dot(pr_ref[0, i], bv, preferred_element_type=jnp.float32)


def _attn_av(probs, sv):
    nb = NCHUNK // _CB
    grid = (HEADS, nb)

    def cur(hd, cb):
        return (hd, cb, 0, 0)

    def prev(hd, cb):
        return (hd, (cb + nb - 1) % nb, 0, 0)

    return _pcall(
        _attn_av_body,
        grid=grid,
        in_specs=[
            pl.BlockSpec((1, _CB, CHUNK, 2 * CHUNK), cur),
            pl.BlockSpec((1, _CB, CHUNK, DIMH), cur),
            pl.BlockSpec((1, _CB, CHUNK, DIMH), prev),
        ],
        out_specs=pl.BlockSpec((1, _CB, CHUNK, DIMH), cur),
        out_shape=jax.ShapeDtypeStruct((HEADS, NCHUNK, CHUNK, DIMH), jnp.float32),
    )(probs, sv, sv)


def _mmadd_body(x_ref, w_ref, b_ref, r_ref, o_ref):
    o_ref[...] = r_ref[...] + (
        jnp.dot(x_ref[...], w_ref[...], preferred_element_type=jnp.float32)
        + b_ref[...])


def _mmadd(x, w, b, res):
    blk = 256
    n = x.shape[0]
    ko, no = w.shape
    return _pcall(
        _mmadd_body,
        grid=(n // blk,),
        in_specs=[
            pl.BlockSpec((blk, ko), lambda i: (i, 0)),
            pl.BlockSpec((ko, no), lambda i: (0, 0)),
            pl.BlockSpec((no,), lambda i: (0,)),
            pl.BlockSpec((blk, no), lambda i: (i, 0)),
        ],
        out_specs=pl.BlockSpec((blk, no), lambda i: (i, 0)),
        out_shape=jax.ShapeDtypeStruct((n, no), jnp.float32),
    )(x, w, b, res)


def _mmb_body(x_ref, w_ref, b_ref, o_ref):
    o_ref[...] = (jnp.dot(x_ref[...], w_ref[...], preferred_element_type=jnp.float32)
                  + b_ref[...])


def _mmb(x, w, b):
    blk = 256
    n = x.shape[0]
    ko, no = w.shape
    return _pcall(
        _mmb_body,
        grid=(n // blk,),
        in_specs=[
            pl.BlockSpec((blk, ko), lambda i: (i, 0)),
            pl.BlockSpec((ko, no), lambda i: (0, 0)),
            pl.BlockSpec((no,), lambda i: (0,)),
        ],
        out_specs=pl.BlockSpec((blk, no), lambda i: (i, 0)),
        out_shape=jax.ShapeDtypeStruct((n, no), jnp.float32),
    )(x, w, b)


def _logits_body(x1_ref, x2_ref, w_ref, b_ref, out_ref):
    x = x1_ref[...] + x2_ref[...]
    out_ref[...] = (jnp.dot(x, w_ref[...], preferred_element_type=jnp.float32)
                    + b_ref[...])


def _logits(x1, x2, w, b):
    nt = 1280
    grid = (NTOK // nt,)
    return _pcall(
        _logits_body,
        grid=grid,
        in_specs=[
            pl.BlockSpec((SEQ, EMB), lambda i: (0, 0)),
            pl.BlockSpec((SEQ, EMB), lambda i: (0, 0)),
            pl.BlockSpec((EMB, nt), lambda i: (0, i)),
            pl.BlockSpec((1, nt), lambda i: (0, i)),
        ],
        out_specs=pl.BlockSpec((SEQ, nt), lambda i: (0, i)),
        out_shape=jax.ShapeDtypeStruct((SEQ, NTOK), jnp.float32),
    )(x1, x2, w, b.reshape(1, NTOK))


def _embed_add_body(e_ref, p_ref, o_ref):
    o_ref[...] = e_ref[...] + p_ref[...]


def _embed_add(e, p):
    return _pcall(
        _embed_add_body,
        grid=(SEQ // 256,),
        in_specs=[pl.BlockSpec((256, EMB), lambda i: (i, 0)),
                  pl.BlockSpec((256, EMB), lambda i: (i, 0))],
        out_specs=pl.BlockSpec((256, EMB), lambda i: (i, 0)),
        out_shape=jax.ShapeDtypeStruct((SEQ, EMB), jnp.float32),
    )(e, p)


_NC, _NS = 2, 16
_NW = _NC * _NS
_SCCHUNK = 512
_CPW = (NITEM // _SCCHUNK) // _NW
_PPC = SEQ // _SCCHUNK
_EPW = SEQ // _NW


def _sc_wid():
    return lax.axis_index("s") * _NC + lax.axis_index("c")


def _sc_chunk_coords(k):
    hh = k // (HEADS * _PPC)
    rem = k % (HEADS * _PPC)
    hd = rem // _PPC
    p0 = (rem % _PPC) * _SCCHUNK
    return hh, hd, p0


@functools.cache
def _get_sc_embed():
    mesh = plsc.VectorSubcoreMesh(core_axis_name="c", subcore_axis_name="s")

    @functools.partial(
        pl.kernel,
        out_type=jax.ShapeDtypeStruct((SEQ, EMB), jnp.float32),
        mesh=mesh,
        compiler_params=pltpu.CompilerParams(use_tc_tiling_on_sc=False),
        scratch_types=[pltpu.VMEM((_EPW,), jnp.int32),
                       pltpu.VMEM((_EPW, EMB), jnp.float32),
                       pltpu.SemaphoreType.DMA],
    )
    def sc_embed(tok_hbm, table_hbm, out_hbm, idx_v, rows, sem):
        wid = _sc_wid()
        base = wid * _EPW
        pltpu.sync_copy(tok_hbm.at[pl.ds(base, _EPW)], idx_v)
        pltpu.async_copy(table_hbm.at[idx_v], rows, sem).wait()
        pltpu.sync_copy(rows, out_hbm.at[pl.ds(base, _EPW), :])

    return sc_embed


@functools.cache
def _get_sc_scatter():
    mesh = plsc.VectorSubcoreMesh(core_axis_name="c", subcore_axis_name="s")

    @functools.partial(
        pl.kernel,
        out_type=[jax.ShapeDtypeStruct((NITEM, DIMH), jnp.float32),
                  jax.ShapeDtypeStruct((NITEM, DIMH), jnp.float32),
                  jax.ShapeDtypeStruct((NITEM, 16), jnp.float32)],
        mesh=mesh,
        compiler_params=pltpu.CompilerParams(use_tc_tiling_on_sc=False),
        scratch_types=[pltpu.VMEM((_CPW * 4, 128), jnp.int32),
                       pltpu.VMEM((_SCCHUNK, DIMH), jnp.float32),
                       pltpu.VMEM((_SCCHUNK, DIMH), jnp.float32),
                       pltpu.VMEM((_SCCHUNK, 16), jnp.float32),
                       pltpu.SemaphoreType.DMA,
                       pltpu.SemaphoreType.DMA,
                       pltpu.SemaphoreType.DMA],
    )
    def sc_scatter(qk_hbm, v_hbm, pos_hbm, idx_hbm, sqk_hbm, sv_hbm, spos_hbm,
                   idx_v, qrows, vrows, prows, semq, semv, semp):
        wid = _sc_wid()
        pltpu.sync_copy(idx_hbm.at[pl.ds(wid * (_CPW * 4), _CPW * 4), :], idx_v)
        for j in range(_CPW):
            k = wid * _CPW + j
            _, hd, p0 = _sc_chunk_coords(k)
            pltpu.sync_copy(qk_hbm.at[hd, pl.ds(p0, _SCCHUNK), :], qrows)
            pltpu.sync_copy(v_hbm.at[hd, pl.ds(p0, _SCCHUNK), :], vrows)
            pltpu.sync_copy(pos_hbm.at[pl.ds(p0, _SCCHUNK), :], prows)
            cps = []
            for i in range(4):
                idx_row = idx_v.at[j * 4 + i]
                cps.append(pltpu.async_copy(
                    qrows.at[pl.ds(i * 128, 128), :], sqk_hbm.at[idx_row], semq))
                cps.append(pltpu.async_copy(
                    vrows.at[pl.ds(i * 128, 128), :], sv_hbm.at[idx_row], semv))
                cps.append(pltpu.async_copy(
                    prows.at[pl.ds(i * 128, 128), :], spos_hbm.at[idx_row], semp))
            for c in cps:
                c.wait()

    return sc_scatter


@functools.cache
def _get_sc_gather():
    mesh = plsc.VectorSubcoreMesh(core_axis_name="c", subcore_axis_name="s")

    @functools.partial(
        pl.kernel,
        out_type=[jax.ShapeDtypeStruct((HEADS, NHASH, SEQ, DIMH), jnp.float32),
                  jax.ShapeDtypeStruct((HEADS, NHASH, SEQ, 16), jnp.float32)],
        mesh=mesh,
        compiler_params=pltpu.CompilerParams(use_tc_tiling_on_sc=False),
        scratch_types=[pltpu.VMEM((_CPW * 4, 128), jnp.int32),
                       pltpu.VMEM((_SCCHUNK, DIMH), jnp.float32),
                       pltpu.VMEM((_SCCHUNK, 16), jnp.float32),
                       pltpu.SemaphoreType.DMA,
                       pltpu.SemaphoreType.DMA],
    )
    def sc_gather(so_hbm, lg_hbm, idx_hbm, og_hbm, lg16_hbm,
                  idx_v, orows, lrows, semo, seml):
        wid = _sc_wid()
        pltpu.sync_copy(idx_hbm.at[pl.ds(wid * (_CPW * 4), _CPW * 4), :], idx_v)
        for j in range(_CPW):
            k = wid * _CPW + j
            hh, hd, p0 = _sc_chunk_coords(k)
            cps = []
            for i in range(4):
                idx_row = idx_v.at[j * 4 + i]
                cps.append(pltpu.async_copy(
                    so_hbm.at[idx_row], orows.at[pl.ds(i * 128, 128), :], semo))
                cps.append(pltpu.async_copy(
                    lg_hbm.at[idx_row], lrows.at[pl.ds(i * 128, 128), :], seml))
            for c in cps:
                c.wait()
            pltpu.sync_copy(orows, og_hbm.at[hd, hh, pl.ds(p0, _SCCHUNK), :])
            pltpu.sync_copy(lrows, lg16_hbm.at[hd, hh, pl.ds(p0, _SCCHUNK), :])

    return sc_gather


def _scatter_sorted(qk, v, gdest):
    qk_h = qk.reshape(SEQ, HEADS, DIMH).transpose(1, 0, 2)
    v_h = v.reshape(SEQ, HEADS, DIMH).transpose(1, 0, 2)
    slot = gdest.reshape(HEADS, TOTAL) - (jnp.arange(HEADS) * TOTAL)[:, None]
    qk4 = jnp.broadcast_to(qk_h[:, None], (HEADS, NHASH, SEQ, DIMH)).reshape(HEADS, TOTAL, DIMH)
    v4 = jnp.broadcast_to(v_h[:, None], (HEADS, NHASH, SEQ, DIMH)).reshape(HEADS, TOTAL, DIMH)
    pos4 = jnp.broadcast_to(jnp.arange(SEQ, dtype=jnp.float32)[None, None, :],
                            (HEADS, NHASH, SEQ)).reshape(HEADS, TOTAL)
    hidx = jnp.arange(HEADS)[:, None]
    sqk = jnp.zeros((HEADS, TOTAL, DIMH), jnp.float32).at[hidx, slot].set(qk4)
    sv = jnp.zeros((HEADS, TOTAL, DIMH), jnp.float32).at[hidx, slot].set(v4)
    sp = jnp.zeros((HEADS, TOTAL), jnp.float32).at[hidx, slot].set(pos4)
    spos = jnp.broadcast_to(sp[..., None], (HEADS, TOTAL, 16))
    return sqk, sv, spos, slot


def _gather_unsorted(so, lg, slot):
    o4 = jnp.take_along_axis(so, slot[..., None], axis=1)
    l4 = jnp.take_along_axis(lg, slot, axis=1)
    return (o4.reshape(HEADS, NHASH, SEQ, DIMH),
            l4.reshape(HEADS, NHASH, SEQ))


def kernel(inputs, params):
    tokens = inputs.reshape(SEQ).astype(jnp.int32)
    emb = _get_sc_embed()(tokens, params['tok_emb'])
    h = _embed_add(emb, params['pos_emb'])
    x1 = h
    x2 = h
    for lp in params['layers']:
        xn = _layer_norm(x2[None], lp['f_g'], lp['f_b'])[0]
        qk, v = _qkv(xn, lp['wqk'], lp['wv'])
        rot2 = lp['rot'].transpose(1, 0, 2)
        gdest = _dest(qk, rot2).reshape(NHASH, HEADS, SEQ).transpose(1, 0, 2)
        sqk, sv, spos, slot = _scatter_sorted(qk, v, gdest)
        sqk4 = sqk.reshape(HEADS, NCHUNK, CHUNK, DIMH)
        sv4 = sv.reshape(HEADS, NCHUNK, CHUNK, DIMH)
        sk4 = sqk4 / (jnp.linalg.norm(sqk4, axis=-1, keepdims=True) + 1e-8)
        dots = _attn_dots(sqk4, sk4, spos.reshape(HEADS, NCHUNK, CHUNK, 16))
        lse = jax.scipy.special.logsumexp(dots, axis=-1, keepdims=True)
        probs = jnp.exp(dots - lse)
        so = _attn_av(probs, sv4)
        o4, l4 = _gather_unsorted(so.reshape(HEADS, TOTAL, DIMH),
                                  lse.reshape(HEADS, TOTAL), slot)
        logits_h = l4[..., None]
        probs_h = jnp.exp(logits_h - jax.scipy.special.logsumexp(
            logits_h, axis=1, keepdims=True))
        o_comb = jnp.sum(o4 * probs_h, axis=1)
        attn_in = o_comb.reshape(1, HEADS, SEQ, DIMH).transpose(0, 2, 1, 3).reshape(SEQ, EMB)
        y1 = _mmadd(attn_in, lp['wo'], lp['bo'], x1)
        xg = _layer_norm(y1[None], lp['g_g'], lp['g_b'])[0]
        hmid = jax.nn.gelu(_mmb(xg, lp['w1'], lp['b1']))
        y2 = _mmadd(hmid, lp['w2'], lp['b2'], x2)
        x1, x2 = y1, y2
    out = _logits(x1, x2, params['w_logits'], params['b_logits'])
    return out.reshape(1, SEQ, NTOK)
